# baseline (speedup 1.0000x reference)
import functools

import jax
import jax.numpy as jnp
from jax import lax
from jax.experimental import pallas as pl
from jax.experimental.pallas import tpu as pltpu
from jax.experimental.pallas import tpu_sc as plsc

N = 10000
D = 256
NEG_SLOPE = 0.2

NC, NS, L = 2, 16, 16
NW = NC * NS
NPT = 320
NPAD = NW * NPT
ROWS = 512
K = 160
OB = 64
NCH = D // L
OFFV = 344
OFFPAD = (NW - 1) * NPT + OFFV
NEG_BIG = -1e30


def _mm2_body(x_ref, wl_ref, wr_ref, xl_ref, xr_ref):
    x = x_ref[...]
    xl_ref[...] = jnp.dot(x, wl_ref[...], preferred_element_type=jnp.float32)
    xr_ref[...] = jnp.dot(x, wr_ref[...], preferred_element_type=jnp.float32)


def _mm2(x, wl, wr):
    return pl.pallas_call(
        _mm2_body,
        grid=(NPAD // ROWS,),
        in_specs=[
            pl.BlockSpec((ROWS, D), lambda i: (i, 0)),
            pl.BlockSpec((D, D), lambda i: (0, 0)),
            pl.BlockSpec((D, D), lambda i: (0, 0)),
        ],
        out_specs=[
            pl.BlockSpec((ROWS, D), lambda i: (i, 0)),
            pl.BlockSpec((ROWS, D), lambda i: (i, 0)),
        ],
        out_shape=[
            jax.ShapeDtypeStruct((NPAD, D), jnp.float32),
            jax.ShapeDtypeStruct((NPAD, D), jnp.float32),
        ],
    )(x, wl, wr)


def _mm2c_body(y_ref, hp_ref, wl_ref, wr_ref, h_ref, xl_ref, xr_ref):
    h = (y_ref[...] + hp_ref[...]) * 0.5
    h_ref[...] = h
    xl_ref[...] = jnp.dot(h, wl_ref[...], preferred_element_type=jnp.float32)
    xr_ref[...] = jnp.dot(h, wr_ref[...], preferred_element_type=jnp.float32)


def _mm2c(y, hp, wl, wr):
    return pl.pallas_call(
        _mm2c_body,
        grid=(NPAD // ROWS,),
        in_specs=[
            pl.BlockSpec((ROWS, D), lambda i: (i, 0)),
            pl.BlockSpec((ROWS, D), lambda i: (i, 0)),
            pl.BlockSpec((D, D), lambda i: (0, 0)),
            pl.BlockSpec((D, D), lambda i: (0, 0)),
        ],
        out_specs=[
            pl.BlockSpec((ROWS, D), lambda i: (i, 0)),
            pl.BlockSpec((ROWS, D), lambda i: (i, 0)),
            pl.BlockSpec((ROWS, D), lambda i: (i, 0)),
        ],
        out_shape=[
            jax.ShapeDtypeStruct((NPAD, D), jnp.float32),
            jax.ShapeDtypeStruct((NPAD, D), jnp.float32),
            jax.ShapeDtypeStruct((NPAD, D), jnp.float32),
        ],
    )(y, hp, wl, wr)


def _edge_body(xl_hbm, xr_hbm, src_hbm, off_hbm, att_hbm, b_hbm, out_hbm,
               idx0, idx1, rows0, rows1, offv, xrg, outb, attv, bv,
               sem0, sem1):
    wid = lax.axis_index("s") * NC + lax.axis_index("c")
    n0 = pl.multiple_of(wid * NPT, NPT)
    pltpu.sync_copy(off_hbm.at[pl.ds(n0, OFFV)], offv)
    pltpu.sync_copy(att_hbm, attv)
    pltpu.sync_copy(b_hbm, bv)
    pltpu.sync_copy(xr_hbm.at[pl.ds(n0, OB)], xrg)
    nreal = jnp.minimum(jnp.int32(NPT), jnp.int32(N) - n0)
    e0 = offv[pl.ds(0, L)][0]
    e1 = offv[pl.ds(nreal, L)][0]
    c0 = lax.div(e0, K)
    c1 = lax.div(e1 + (K - 1), K)
    nc = c1 - c0
    att_regs = [attv[pl.ds(L * c, L)] for c in range(NCH)]
    zero = jnp.zeros((L,), jnp.float32)
    neg = jnp.full((L,), NEG_BIG, jnp.float32)
    sems = (sem0, sem1)
    idxs = (idx0, idx1)
    rows = (rows0, rows1)

    cb0 = pl.multiple_of(c0 * K, K)
    pltpu.sync_copy(src_hbm.at[pl.ds(cb0, K)], idx0)
    pltpu.async_copy(xl_hbm.at[idx0], rows0, sem0)

    xr0 = [xrg[0, pl.ds(L * c, L)] for c in range(NCH)]

    def chunk_compute(ci, b, carry):
        cb = (c0 + ci) * K
        jstart = jnp.maximum(e0 - cb, 0)
        jhi = jnp.minimum(e1 - cb, K)

        def seg_cond(st):
            return (st[0] < jhi) & (st[1] < n0 + nreal)

        def seg_body(st):
            j, cur, mv, lv = st[0], st[1], st[2], st[3]
            acc = list(st[4:4 + NCH])
            xr_regs = list(st[4 + NCH:])
            se = offv[pl.ds(cur + 1 - n0, L)][0]
            ee = jnp.minimum(se - cb, jhi)

            def eb(jj, ec):
                mv, lv = ec[0], ec[1]
                acc = ec[2:]
                p = zero
                xs = []
                for c in range(NCH):
                    xlc = rows[b][jj, pl.ds(L * c, L)]
                    s = xlc + xr_regs[c]
                    s = jnp.maximum(s, s * NEG_SLOPE)
                    p = p + s * att_regs[c]
                    xs.append(xlc)
                ev = jnp.full((L,), jnp.sum(p), jnp.float32)
                mn = jnp.maximum(mv, ev)
                sc = jnp.exp(mv - mn)
                w = jnp.exp(ev - mn)
                lv2 = lv * sc + w
                acc = [a * sc + w * x for a, x in zip(acc, xs)]
                return (mn, lv2, *acc)

            mv, lv, *acc = plsc.parallel_loop(
                j, ee, unroll=4, carry=(mv, lv, *acc))(eb)

            def do_fin(args):
                cur, lv = args[0], args[1]
                acc = args[2:2 + NCH]
                lc = cur - n0
                slot = lax.rem(lc, OB)
                inv = 1.0 / lv
                for c in range(NCH):
                    outb[slot, pl.ds(L * c, L)] = (
                        acc[c] * inv + bv[pl.ds(L * c, L)])

                @pl.when(slot == OB - 1)
                def _():
                    fb = pl.multiple_of(n0 + lc - (OB - 1), OB)
                    pltpu.sync_copy(outb, out_hbm.at[pl.ds(fb, OB)])
                    gb = pl.multiple_of(n0 + lc + 1, OB)
                    pltpu.sync_copy(xr_hbm.at[pl.ds(gb, OB)], xrg)

                curn = cur + 1
                srow = lax.rem(curn - n0, OB)
                xr_new = [xrg[srow, pl.ds(L * c, L)] for c in range(NCH)]
                return (curn, neg, zero, *([zero] * NCH), *xr_new)

            def no_fin(args):
                cur, lv = args[0], args[1]
                acc = args[2:2 + NCH]
                xr_regs = args[2 + NCH:]
                return (cur, mv, lv, *acc, *xr_regs)

            fin = se - cb <= jhi
            cur, mv, lv, *rest = lax.cond(
                fin, do_fin, no_fin, (cur, lv, *acc, *xr_regs))
            return (ee, cur, mv, lv, *rest)

        return lax.while_loop(seg_cond, seg_body, (jstart, *carry))[1:]

    def pair_body(pi, carry):
        for b in (0, 1):
            ci = 2 * pi + b
            cbn = pl.multiple_of((c0 + ci + 1) * K, K)
            pltpu.sync_copy(src_hbm.at[pl.ds(cbn, K)], idxs[1 - b])
            pltpu.async_copy(xl_hbm.at[idxs[1 - b]], rows[1 - b],
                             sems[1 - b])
            pltpu.make_async_copy(xl_hbm.at[idxs[b]], rows[b],
                                  sems[b]).wait()
            carry = lax.cond(ci < nc,
                             lambda cr, ci=ci, b=b: chunk_compute(ci, b, cr),
                             lambda cr: cr, carry)
        return carry

    init = (jnp.int32(0) + n0, neg, zero) + tuple([zero] * NCH) + tuple(xr0)
    npairs = lax.div(nc + 1, 2)
    _ = lax.fori_loop(0, npairs, pair_body, init)

    pltpu.make_async_copy(xl_hbm.at[idx0], rows0, sem0).wait()

    lcl = nreal - 1
    fb = pl.multiple_of(n0 + lcl - lax.rem(lcl, OB), OB)
    pltpu.sync_copy(outb, out_hbm.at[pl.ds(fb, OB)])


@functools.cache
def _edge_sc():
    return pl.kernel(
        _edge_body,
        out_type=jax.ShapeDtypeStruct((NPAD, D), jnp.float32),
        mesh=plsc.VectorSubcoreMesh(core_axis_name="c", subcore_axis_name="s",
                                    num_cores=NC, num_subcores=NS),
        scratch_types=[
            pltpu.VMEM((K,), jnp.int32),
            pltpu.VMEM((K,), jnp.int32),
            pltpu.VMEM((K, D), jnp.float32),
            pltpu.VMEM((K, D), jnp.float32),
            pltpu.VMEM((OFFV,), jnp.int32),
            pltpu.VMEM((OB, D), jnp.float32),
            pltpu.VMEM((OB, D), jnp.float32),
            pltpu.VMEM((D,), jnp.float32),
            pltpu.VMEM((D,), jnp.float32),
            pltpu.SemaphoreType.DMA,
            pltpu.SemaphoreType.DMA,
        ],
        compiler_params=pltpu.CompilerParams(needs_layout_passes=False),
    )


def kernel(input, edge_index, params):
    etot = edge_index.shape[1] + N
    epad = ((etot + K - 1) // K) * K + 2 * K
    loops = jnp.arange(N, dtype=edge_index.dtype)
    src = jnp.concatenate([edge_index[0], loops])
    dst = jnp.concatenate([edge_index[1], loops])
    order = jnp.argsort(dst)
    s_src = src[order].astype(jnp.int32)
    s_dst = dst[order].astype(jnp.int32)
    offsets = jnp.searchsorted(
        s_dst, jnp.arange(OFFPAD, dtype=jnp.int32)).astype(jnp.int32)
    s_src = jnp.concatenate(
        [s_src, jnp.zeros((epad - etot,), jnp.int32)])
    xpad = jnp.concatenate(
        [input, jnp.zeros((NPAD - N, D), jnp.float32)])

    def edge_stage(xl, xr, att, b):
        return _edge_sc()(xl, xr, s_src, offsets, att, b)

    def conv_plain(h, p):
        wl, wr, att, b = p
        xl, xr = _mm2(h, wl, wr)
        return edge_stage(xl, xr, att, b)

    def conv_comb(y, hp, p):
        wl, wr, att, b = p
        h, xl, xr = _mm2c(y, hp, wl, wr)
        return h, edge_stage(xl, xr, att, b)

    h0 = conv_plain(xpad, params[0])
    y = conv_plain(h0, params[1])
    y2 = conv_plain(y, params[2])
    h1, y = conv_comb(y2, h0, params[3])
    y2 = conv_plain(y, params[4])
    h2, y = conv_comb(y2, h1, params[5])
    y2 = conv_plain(y, params[6])
    x_cat, x_out = conv_comb(y2, h2, params[7])
    return (x_out[:N], x_cat[:N])


# speedup vs baseline: 1.0188x; 1.0188x over previous
import functools

import jax
import jax.numpy as jnp
from jax import lax
from jax.experimental import pallas as pl
from jax.experimental.pallas import tpu as pltpu
from jax.experimental.pallas import tpu_sc as plsc

N = 10000
D = 256
NEG_SLOPE = 0.2

NC, NS, L = 2, 16, 16
NW = NC * NS
NPT = 320
NPAD = NW * NPT
ROWS = 512
K = 160
OB = 64
NCH = D // L
OFFV = 344
OFFPAD = (NW - 1) * NPT + OFFV
NEG_BIG = -1e30


def _mm2_body(x_ref, wl_ref, wr_ref, xl_ref, xr_ref):
    x = x_ref[...]
    xl_ref[...] = jnp.dot(x, wl_ref[...], preferred_element_type=jnp.float32)
    xr_ref[...] = jnp.dot(x, wr_ref[...], preferred_element_type=jnp.float32)


def _mm2(x, wl, wr):
    return pl.pallas_call(
        _mm2_body,
        grid=(NPAD // ROWS,),
        in_specs=[
            pl.BlockSpec((ROWS, D), lambda i: (i, 0)),
            pl.BlockSpec((D, D), lambda i: (0, 0)),
            pl.BlockSpec((D, D), lambda i: (0, 0)),
        ],
        out_specs=[
            pl.BlockSpec((ROWS, D), lambda i: (i, 0)),
            pl.BlockSpec((ROWS, D), lambda i: (i, 0)),
        ],
        out_shape=[
            jax.ShapeDtypeStruct((NPAD, D), jnp.float32),
            jax.ShapeDtypeStruct((NPAD, D), jnp.float32),
        ],
    )(x, wl, wr)


def _mm2c_body(y_ref, hp_ref, wl_ref, wr_ref, h_ref, xl_ref, xr_ref):
    h = (y_ref[...] + hp_ref[...]) * 0.5
    h_ref[...] = h
    xl_ref[...] = jnp.dot(h, wl_ref[...], preferred_element_type=jnp.float32)
    xr_ref[...] = jnp.dot(h, wr_ref[...], preferred_element_type=jnp.float32)


def _mm2c(y, hp, wl, wr):
    return pl.pallas_call(
        _mm2c_body,
        grid=(NPAD // ROWS,),
        in_specs=[
            pl.BlockSpec((ROWS, D), lambda i: (i, 0)),
            pl.BlockSpec((ROWS, D), lambda i: (i, 0)),
            pl.BlockSpec((D, D), lambda i: (0, 0)),
            pl.BlockSpec((D, D), lambda i: (0, 0)),
        ],
        out_specs=[
            pl.BlockSpec((ROWS, D), lambda i: (i, 0)),
            pl.BlockSpec((ROWS, D), lambda i: (i, 0)),
            pl.BlockSpec((ROWS, D), lambda i: (i, 0)),
        ],
        out_shape=[
            jax.ShapeDtypeStruct((NPAD, D), jnp.float32),
            jax.ShapeDtypeStruct((NPAD, D), jnp.float32),
            jax.ShapeDtypeStruct((NPAD, D), jnp.float32),
        ],
    )(y, hp, wl, wr)


def _edge_body(xl_hbm, xr_hbm, src_hbm, off_hbm, att_hbm, b_hbm, out_hbm,
               idx0, idx1, rows0, rows1, offv, xrg, outb, attv, bv,
               sem0, sem1):
    wid = lax.axis_index("s") * NC + lax.axis_index("c")
    n0 = pl.multiple_of(wid * NPT, NPT)
    pltpu.sync_copy(off_hbm.at[pl.ds(n0, OFFV)], offv)
    pltpu.sync_copy(att_hbm, attv)
    pltpu.sync_copy(b_hbm, bv)
    pltpu.sync_copy(xr_hbm.at[pl.ds(n0, OB)], xrg)
    nreal = jnp.minimum(jnp.int32(NPT), jnp.int32(N) - n0)
    e0 = offv[pl.ds(0, L)][0]
    e1 = offv[pl.ds(nreal, L)][0]
    c0 = lax.div(e0, K)
    c1 = lax.div(e1 + (K - 1), K)
    nc = c1 - c0
    att_regs = [attv[pl.ds(L * c, L)] for c in range(NCH)]
    zero = jnp.zeros((L,), jnp.float32)
    neg = jnp.full((L,), NEG_BIG, jnp.float32)
    sems = (sem0, sem1)
    idxs = (idx0, idx1)
    rows = (rows0, rows1)

    cb0 = pl.multiple_of(c0 * K, K)
    pltpu.sync_copy(src_hbm.at[pl.ds(cb0, K)], idx0)
    pltpu.async_copy(xl_hbm.at[idx0], rows0, sem0)

    xr0 = [xrg[0, pl.ds(L * c, L)] for c in range(NCH)]

    def chunk_compute(ci, b, carry):
        cb = (c0 + ci) * K
        jstart = jnp.maximum(e0 - cb, 0)
        jhi = jnp.minimum(e1 - cb, K)

        def seg_cond(st):
            return (st[0] < jhi) & (st[1] < n0 + nreal)

        def seg_body(st):
            j, cur, mv, lv = st[0], st[1], st[2], st[3]
            acc = list(st[4:4 + NCH])
            xr_regs = list(st[4 + NCH:])
            se = offv[pl.ds(cur + 1 - n0, L)][0]
            ee = jnp.minimum(se - cb, jhi)

            def eb(jj, ec):
                mv, lv = ec[0], ec[1]
                acc = ec[2:]
                p = zero
                xs = []
                for c in range(NCH):
                    xlc = rows[b][jj, pl.ds(L * c, L)]
                    s = xlc + xr_regs[c]
                    s = jnp.maximum(s, s * NEG_SLOPE)
                    p = p + s * att_regs[c]
                    xs.append(xlc)
                ev = jnp.full((L,), jnp.sum(p), jnp.float32)
                mn = jnp.maximum(mv, ev)
                sc = jnp.exp(mv - mn)
                w = jnp.exp(ev - mn)
                lv2 = lv * sc + w
                acc = [a * sc + w * x for a, x in zip(acc, xs)]
                return (mn, lv2, *acc)

            mv, lv, *acc = plsc.parallel_loop(
                j, ee, unroll=2, carry=(mv, lv, *acc))(eb)

            def do_fin(args):
                cur, lv = args[0], args[1]
                acc = args[2:2 + NCH]
                lc = cur - n0
                slot = lax.rem(lc, OB)
                inv = 1.0 / lv
                for c in range(NCH):
                    outb[slot, pl.ds(L * c, L)] = (
                        acc[c] * inv + bv[pl.ds(L * c, L)])

                @pl.when(slot == OB - 1)
                def _():
                    fb = pl.multiple_of(n0 + lc - (OB - 1), OB)
                    pltpu.sync_copy(outb, out_hbm.at[pl.ds(fb, OB)])
                    gb = pl.multiple_of(n0 + lc + 1, OB)
                    pltpu.sync_copy(xr_hbm.at[pl.ds(gb, OB)], xrg)

                curn = cur + 1
                srow = lax.rem(curn - n0, OB)
                xr_new = [xrg[srow, pl.ds(L * c, L)] for c in range(NCH)]
                return (curn, neg, zero, *([zero] * NCH), *xr_new)

            def no_fin(args):
                cur, lv = args[0], args[1]
                acc = args[2:2 + NCH]
                xr_regs = args[2 + NCH:]
                return (cur, mv, lv, *acc, *xr_regs)

            fin = se - cb <= jhi
            cur, mv, lv, *rest = lax.cond(
                fin, do_fin, no_fin, (cur, lv, *acc, *xr_regs))
            return (ee, cur, mv, lv, *rest)

        return lax.while_loop(seg_cond, seg_body, (jstart, *carry))[1:]

    def pair_body(pi, carry):
        for b in (0, 1):
            ci = 2 * pi + b
            cbn = pl.multiple_of((c0 + ci + 1) * K, K)
            pltpu.sync_copy(src_hbm.at[pl.ds(cbn, K)], idxs[1 - b])
            pltpu.async_copy(xl_hbm.at[idxs[1 - b]], rows[1 - b],
                             sems[1 - b])
            pltpu.make_async_copy(xl_hbm.at[idxs[b]], rows[b],
                                  sems[b]).wait()
            carry = lax.cond(ci < nc,
                             lambda cr, ci=ci, b=b: chunk_compute(ci, b, cr),
                             lambda cr: cr, carry)
        return carry

    init = (jnp.int32(0) + n0, neg, zero) + tuple([zero] * NCH) + tuple(xr0)
    npairs = lax.div(nc + 1, 2)
    _ = lax.fori_loop(0, npairs, pair_body, init)

    pltpu.make_async_copy(xl_hbm.at[idx0], rows0, sem0).wait()

    lcl = nreal - 1
    fb = pl.multiple_of(n0 + lcl - lax.rem(lcl, OB), OB)
    pltpu.sync_copy(outb, out_hbm.at[pl.ds(fb, OB)])


@functools.cache
def _edge_sc():
    return pl.kernel(
        _edge_body,
        out_type=jax.ShapeDtypeStruct((NPAD, D), jnp.float32),
        mesh=plsc.VectorSubcoreMesh(core_axis_name="c", subcore_axis_name="s",
                                    num_cores=NC, num_subcores=NS),
        scratch_types=[
            pltpu.VMEM((K,), jnp.int32),
            pltpu.VMEM((K,), jnp.int32),
            pltpu.VMEM((K, D), jnp.float32),
            pltpu.VMEM((K, D), jnp.float32),
            pltpu.VMEM((OFFV,), jnp.int32),
            pltpu.VMEM((OB, D), jnp.float32),
            pltpu.VMEM((OB, D), jnp.float32),
            pltpu.VMEM((D,), jnp.float32),
            pltpu.VMEM((D,), jnp.float32),
            pltpu.SemaphoreType.DMA,
            pltpu.SemaphoreType.DMA,
        ],
        compiler_params=pltpu.CompilerParams(needs_layout_passes=False),
    )


def kernel(input, edge_index, params):
    etot = edge_index.shape[1] + N
    epad = ((etot + K - 1) // K) * K + 2 * K
    loops = jnp.arange(N, dtype=edge_index.dtype)
    src = jnp.concatenate([edge_index[0], loops])
    dst = jnp.concatenate([edge_index[1], loops])
    s_dst, s_src = lax.sort((dst, src), num_keys=1)
    s_src = s_src.astype(jnp.int32)
    s_dst = s_dst.astype(jnp.int32)
    offsets = jnp.searchsorted(
        s_dst, jnp.arange(OFFPAD, dtype=jnp.int32)).astype(jnp.int32)
    s_src = jnp.concatenate(
        [s_src, jnp.zeros((epad - etot,), jnp.int32)])
    xpad = jnp.concatenate(
        [input, jnp.zeros((NPAD - N, D), jnp.float32)])

    def edge_stage(xl, xr, att, b):
        return _edge_sc()(xl, xr, s_src, offsets, att, b)

    def conv_plain(h, p):
        wl, wr, att, b = p
        xl, xr = _mm2(h, wl, wr)
        return edge_stage(xl, xr, att, b)

    def conv_comb(y, hp, p):
        wl, wr, att, b = p
        h, xl, xr = _mm2c(y, hp, wl, wr)
        return h, edge_stage(xl, xr, att, b)

    h0 = conv_plain(xpad, params[0])
    y = conv_plain(h0, params[1])
    y2 = conv_plain(y, params[2])
    h1, y = conv_comb(y2, h0, params[3])
    y2 = conv_plain(y, params[4])
    h2, y = conv_comb(y2, h1, params[5])
    y2 = conv_plain(y, params[6])
    x_cat, x_out = conv_comb(y2, h2, params[7])
    return (x_out[:N], x_cat[:N])
